# baseline (speedup 1.0000x reference)
import functools

import jax
import jax.numpy as jnp
import numpy as np
from jax import lax
from jax.experimental import pallas as pl
from jax.experimental.pallas import tpu as pltpu
from jax.experimental.pallas import tpu_sc as plsc

_H = 8
_N = 38416
_NV = _N // 16
_NW = 32
_VPW = 76
_CHUNK = _VPW * 16


def _bb_pos_flat(qq, kk):
    strides = int(np.ceil(np.sqrt(float(kk) / float(qq))))
    q_h = int(np.sqrt(float(qq)))
    k_h = int(np.sqrt(float(kk)))
    x1, y1 = np.meshgrid(np.arange(q_h), np.arange(q_h))
    x2, y2 = np.meshgrid(np.arange(k_h), np.arange(k_h))
    aa = np.concatenate([x1.reshape(-1, 1), y1.reshape(-1, 1)], axis=-1)
    b2 = np.concatenate([x2.reshape(-1, 1), y2.reshape(-1, 1)], axis=-1)
    cc = np.abs(b2[None, :, :] - aa[:, None, :] * strides)
    pos = cc[:, :, 0] + cc[:, :, 1] * k_h
    return pos.reshape(-1).astype(np.int32)


def _sc_gather_body(bb_hbm, idx_hbm, out_hbm, idx_v, bb_v, res_v):
    wid = lax.axis_index("s") * 2 + lax.axis_index("c")
    base = jnp.minimum(wid * _CHUNK, _N - _CHUNK)
    pltpu.sync_copy(bb_hbm, bb_v)
    pltpu.sync_copy(idx_hbm.at[pl.ds(base, _CHUNK)], idx_v)

    def j_body(j, carry):
        idx8 = idx_v[pl.ds(j * 16, 16)] * 8
        for h in range(_H):
            res_v[pl.ds(h * _CHUNK + j * 16, 16)] = plsc.load_gather(bb_v, [idx8 + h])
        return carry

    lax.fori_loop(0, _VPW, j_body, 0)
    for h in range(_H):
        pltpu.sync_copy(res_v.at[pl.ds(h * _CHUNK, _CHUNK)],
                        out_hbm.at[pl.ds(h * _N + base, _CHUNK)])


def _sc_gather(bb, idx_flat):
    mesh = plsc.VectorSubcoreMesh(core_axis_name="c", subcore_axis_name="s")
    return pl.kernel(
        _sc_gather_body,
        mesh=mesh,
        compiler_params=pltpu.CompilerParams(needs_layout_passes=False),
        out_type=jax.ShapeDtypeStruct((_H * _N,), jnp.float32),
        scratch_types=[
            pltpu.VMEM((_CHUNK,), jnp.int32),
            pltpu.VMEM((196 * _H,), jnp.float32),
            pltpu.VMEM((_H * _CHUNK,), jnp.float32),
        ],
    )(bb, idx_flat)


def _add_body(x_ref, pb_ref, o_ref):
    o_ref[...] = x_ref[...] + pb_ref[...]


def _tc_add(x3, pb):
    B = x3.shape[0]
    bblk = 8
    return pl.pallas_call(
        _add_body,
        grid=(B // bblk,),
        in_specs=[
            pl.BlockSpec((bblk, _H, _N), lambda b: (b, 0, 0)),
            pl.BlockSpec((1, _H, _N), lambda b: (0, 0, 0)),
        ],
        out_specs=pl.BlockSpec((bblk, _H, _N), lambda b: (b, 0, 0)),
        out_shape=jax.ShapeDtypeStruct((B, _H, _N), jnp.float32),
    )(x3, pb)


@jax.jit
def kernel(inputs, bb):
    B, H, QQ, KK = inputs.shape
    idx_flat = jnp.asarray(_bb_pos_flat(QQ, KK))
    pb = _sc_gather(bb.reshape(-1), idx_flat)
    x3 = inputs.reshape(B, H, QQ * KK)
    out = _tc_add(x3, pb.reshape(1, H, QQ * KK))
    return out.reshape(B, H, QQ, KK)
